# baseline (speedup 1.0000x reference)
import functools

import jax
import jax.numpy as jnp
from jax import lax
from jax.experimental import pallas as pl
from jax.experimental.pallas import tpu as pltpu
from jax.experimental.pallas import tpu_sc as plsc

B, C, H, W, K = 4, 96, 384, 384, 256
HW = H * W
S = K + 1
NLANE = 16
CPT = C // 8
CH_PAR = 6
CG = CPT // CH_PAR
ROWS = 8
P = ROWS * W
NCHUNK = HW // P
VPR = W // NLANE
UNROLL = 2
BREG = NLANE * S
BINS_W = -(-(CG * BREG) // 128) * 128

_mesh = plsc.VectorSubcoreMesh(core_axis_name="c", subcore_axis_name="s")


@functools.partial(
    pl.kernel,
    mesh=_mesh,
    out_type=jax.ShapeDtypeStruct((B * C * K,), jnp.float32),
    scratch_types=(
        [pltpu.VMEM((P,), jnp.int32) for _ in range(2)]
        + [pltpu.VMEM((CH_PAR, ROWS, W), jnp.float32) for _ in range(2)]
        + [pltpu.VMEM((BINS_W,), jnp.float32) for _ in range(CH_PAR)]
        + [pltpu.VMEM((CPT * K,), jnp.float32)]
        + [pltpu.SemaphoreType.DMA for _ in range(2)]
    ),
    compiler_params=pltpu.CompilerParams(needs_layout_passes=False),
)
def _seg_max_kernel(x_hbm, idx_hbm, out_hbm, *refs):
    idx_bufs = refs[0:2]
    data_bufs = refs[2:4]
    bins = refs[4:4 + CH_PAR]
    res_v = refs[4 + CH_PAR]
    sems = refs[5 + CH_PAR:7 + CH_PAR]

    wid = lax.axis_index("s") * 2 + lax.axis_index("c")
    b = wid // 8
    c0 = (wid % 8) * CPT

    neg = jnp.full((NLANE,), -jnp.inf, jnp.float32)

    def init_body(i, _):
        for q in range(CH_PAR):
            bins[q][pl.ds(i * NLANE, NLANE)] = neg
        return 0

    lax.fori_loop(0, BINS_W // NLANE, init_body, 0)

    def copies(st, cg, ch):
        return [
            pltpu.make_async_copy(
                idx_hbm.at[pl.ds(b * HW + ch * P, P)], idx_bufs[st], sems[st]),
            pltpu.make_async_copy(
                x_hbm.at[b, pl.ds(c0 + cg * CH_PAR, CH_PAR),
                         pl.ds(ch * ROWS, ROWS), :],
                data_bufs[st], sems[st]),
        ]

    def start(st, cg, ch):
        for d in copies(st, cg, ch):
            d.start()

    def wait(st, cg, ch):
        for d in copies(st, cg, ch):
            d.wait()

    def compute(st, cg):
        coff = cg * BREG
        dbuf = data_bufs[st]
        ibuf = idx_bufs[st]

        def row_body(r, _):
            for u in range(VPR):
                o = u * NLANE
                vidx = ibuf[pl.ds(r * W + o, NLANE)]
                if coff:
                    vidx = vidx + coff
                vdat = [dbuf[q, r, pl.ds(o, NLANE)] for q in range(CH_PAR)]
                cur = [plsc.load_gather(bins[q], [vidx])
                       for q in range(CH_PAR)]
                for q in range(CH_PAR):
                    plsc.store_scatter(bins[q], [vidx],
                                       jnp.maximum(cur[q], vdat[q]))
            return 0

        lax.fori_loop(0, ROWS, row_body, 0)

    start(0, 0, 0)

    def chunk_body(i, _):
        start(1, 1, i)
        wait(0, 0, i)
        compute(0, 0)

        @pl.when(i + 1 < NCHUNK)
        def _():
            start(0, 0, i + 1)

        wait(1, 1, i)
        compute(1, 1)
        return 0

    lax.fori_loop(0, NCHUNK, chunk_body, 0)

    lane_iota = lax.iota(jnp.int32, NLANE)

    def red_j(j, _):
        for cg in range(CG):
            for q in range(CH_PAR):
                g = cg * BREG + j * NLANE + lane_iota
                acc = plsc.load_gather(bins[q], [g])
                for l in range(1, NLANE):
                    acc = jnp.maximum(acc, plsc.load_gather(bins[q], [g + l * S]))
                cl = cg * CH_PAR + q
                res_v[pl.ds(cl * K + j * NLANE, NLANE)] = acc
        return 0

    lax.fori_loop(0, K // NLANE, red_j, 0)
    pltpu.sync_copy(res_v, out_hbm.at[pl.ds((b * C + c0) * K, CPT * K)])


def kernel(input_features_in, label_mask, device=0):
    lab = label_mask.reshape(B, H, W)
    row = jnp.arange(H, dtype=jnp.int32)[:, None]
    col = jnp.arange(W, dtype=jnp.int32)[None, :]
    border = (row == 0) | (row == H - 1) | (col == 0) | (col == W - 1)
    lab = jnp.where(border[None], K, lab).reshape(B, HW)
    lane = (jnp.arange(HW, dtype=jnp.int32) % NLANE) * S
    idx = (lab + lane[None]).reshape(B * HW)
    return _seg_max_kernel(input_features_in, idx).reshape(B, C, K)
